# baseline (speedup 1.0000x reference)
import functools

import jax
import jax.numpy as jnp
from jax import lax
from jax.experimental import pallas as pl
from jax.experimental.pallas import tpu as pltpu
from jax.experimental.pallas import tpu_sc as plsc

T = 2048
IN_F = 2048
OUT_F = 8192
BLK = 512
NB = OUT_F // BLK
TOPK = NB // 4

RT = 256
NT = (T * TOPK + NB * RT) // RT
P_MAX = NT * RT
NTP = 64

NW = 32
TPW = T // NW
CCH = 16
TPC = 4

_HI = jax.lax.Precision.HIGHEST


def _meta_body(prob_ref, pos_tok_ref, pos_t_ref, te_ref, xi_ref, tv_ref):
    prob = prob_ref[...]
    ids_e = lax.broadcasted_iota(jnp.int32, (T, NB), 1)

    cols = []
    for e in range(NB):
        pn = prob[:, e:e + 1]
        beats = (prob > pn) | ((prob == pn) & (ids_e < e))
        cnt = jnp.sum(beats.astype(jnp.float32), axis=1, keepdims=True)
        cols.append((cnt < TOPK).astype(jnp.float32))
    maskf = jnp.concatenate(cols, axis=1)
    maskb = maskf > 0.5

    r_i = lax.broadcasted_iota(jnp.int32, (T, T), 0)
    c_i = lax.broadcasted_iota(jnp.int32, (T, T), 1)
    tril = (c_i < r_i).astype(jnp.float32)
    rank = lax.dot_general(tril, maskf, (((1,), (0,)), ((), ())),
                           precision=_HI)

    ones_row = jnp.ones((1, T), jnp.float32)
    counts = lax.dot_general(ones_row, maskf, (((1,), (0,)), ((), ())),
                             precision=_HI)
    pc = jnp.floor((counts + (RT - 1)) / RT) * RT

    re = lax.broadcasted_iota(jnp.int32, (NB, NB), 0)
    ce = lax.broadcasted_iota(jnp.int32, (NB, NB), 1)
    l16s = (re < ce).astype(jnp.float32)
    starts = lax.dot_general(pc, l16s, (((1,), (0,)), ((), ())),
                             precision=_HI)
    ends = starts + pc

    p_te = starts + rank

    l16i = (re <= ce).astype(jnp.float32)
    ordm = lax.dot_general(maskf, l16i, (((1,), (0,)), ((), ())),
                           precision=_HI)

    pcols = []
    for j in range(TOPK):
        selj = maskb & (ordm == (j + 1))
        pcols.append(jnp.sum(jnp.where(selj, p_te, 0.0), axis=1, keepdims=True))
    pos_tok = jnp.concatenate(pcols, axis=1)
    pos_tok_ref[...] = pos_tok.astype(jnp.int32)

    ident = (r_i == c_i).astype(jnp.float32)
    pos_t = lax.dot_general(pos_tok, ident, (((0,), (0,)), ((), ())),
                            precision=_HI)
    pos_t_ref[...] = pos_t.astype(jnp.int32)

    u = jnp.sum(pc, axis=1, keepdims=True) / RT
    it = lax.broadcasted_iota(jnp.int32, (NTP, NB), 0).astype(jnp.float32)
    texp_raw = jnp.sum((it * RT >= ends).astype(jnp.float32),
                       axis=1, keepdims=True)
    texp_last = jnp.sum(((u - 1.0) * RT >= ends).astype(jnp.float32),
                        axis=1, keepdims=True)
    itcol = lax.broadcasted_iota(jnp.int32, (NTP, 1), 0).astype(jnp.float32)
    valid = itcol < u
    te_ref[...] = jnp.where(valid, texp_raw, texp_last).astype(jnp.int32)
    xi_ref[...] = jnp.minimum(itcol, u - 1.0).astype(jnp.int32)
    tv_ref[...] = valid.astype(jnp.int32)


def _run_meta(prob):
    return pl.pallas_call(
        _meta_body,
        out_shape=[
            jax.ShapeDtypeStruct((T, TOPK), jnp.int32),
            jax.ShapeDtypeStruct((TOPK, T), jnp.int32),
            jax.ShapeDtypeStruct((NTP, 1), jnp.int32),
            jax.ShapeDtypeStruct((NTP, 1), jnp.int32),
            jax.ShapeDtypeStruct((NTP, 1), jnp.int32),
        ],
    )(prob)


def _scatter_body(x_hbm, pos_t_hbm, xs_hbm, xbuf, idxbuf, sem):
    wid = lax.axis_index("s") * 2 + lax.axis_index("c")
    base = wid * TPW

    def chunk(k, carry):
        t0 = base + k * CCH
        pltpu.sync_copy(x_hbm.at[pl.ds(t0, CCH)], xbuf)
        for j in range(TOPK):
            pltpu.sync_copy(pos_t_hbm.at[j, pl.ds(t0, CCH)], idxbuf)
            pltpu.async_copy(xbuf, xs_hbm.at[idxbuf], sem).wait()
        return carry

    lax.fori_loop(0, TPW // CCH, chunk, 0)


def _run_scatter(x, pos_t):
    mesh = plsc.VectorSubcoreMesh(core_axis_name="c", subcore_axis_name="s")
    f = functools.partial(
        pl.kernel,
        out_type=jax.ShapeDtypeStruct((P_MAX, IN_F), jnp.float32),
        mesh=mesh,
        scratch_types=[
            pltpu.VMEM((CCH, IN_F), jnp.float32),
            pltpu.VMEM((CCH,), jnp.int32),
            pltpu.SemaphoreType.DMA,
        ],
    )(_scatter_body)
    return f(x, pos_t)


def _ffn_body(te_ref, xi_ref, tv_ref, xs_ref, w1_ref, b1_ref, w2_ref, ys_ref):
    i = pl.program_id(0)

    @pl.when(tv_ref[i] == 1)
    def _():
        xt = xs_ref[...]
        h = lax.dot_general(xt, w1_ref[...], (((1,), (1,)), ((), ())),
                            preferred_element_type=jnp.float32)
        h = h + b1_ref[0]
        g = jax.nn.gelu(h)
        ys_ref[...] = lax.dot_general(g, w2_ref[...], (((1,), (1,)), ((), ())),
                                      preferred_element_type=jnp.float32)


def _run_ffn(te, xi, tv, xs, W1, b1r, W2):
    grid_spec = pltpu.PrefetchScalarGridSpec(
        num_scalar_prefetch=3,
        grid=(NT,),
        in_specs=[
            pl.BlockSpec((RT, IN_F), lambda i, te, xi, tv: (xi[i], 0)),
            pl.BlockSpec((BLK, IN_F), lambda i, te, xi, tv: (te[i], 0)),
            pl.BlockSpec((1, 1, BLK), lambda i, te, xi, tv: (te[i], 0, 0)),
            pl.BlockSpec((IN_F, BLK), lambda i, te, xi, tv: (0, te[i])),
        ],
        out_specs=pl.BlockSpec((RT, IN_F), lambda i, te, xi, tv: (xi[i], 0)),
    )
    return pl.pallas_call(
        _ffn_body,
        grid_spec=grid_spec,
        out_shape=jax.ShapeDtypeStruct((P_MAX, IN_F), jnp.float32),
        compiler_params=pltpu.CompilerParams(
            dimension_semantics=("arbitrary",),
        ),
    )(te, xi, tv, xs, W1, b1r, W2)


def _combine_body(ys_hbm, pos_flat_hbm, b2_hbm, y_hbm, rows, acc, idxbuf,
                  b2v, sem):
    wid = lax.axis_index("s") * 2 + lax.axis_index("c")
    base = wid * TPW
    pltpu.sync_copy(b2_hbm, b2v)

    def chunk(k, carry):
        t0 = base + k * TPC
        pltpu.sync_copy(pos_flat_hbm.at[pl.ds(t0 * TOPK, TPC * TOPK)], idxbuf)
        pltpu.async_copy(ys_hbm.at[idxbuf], rows, sem).wait()

        def col(ci, carry2):
            off = ci * 16
            for i in range(TPC):
                v = b2v[pl.ds(off, 16)]
                for j in range(TOPK):
                    v = v + rows[TOPK * i + j, pl.ds(off, 16)]
                acc[i, pl.ds(off, 16)] = v
            return carry2

        lax.fori_loop(0, IN_F // 16, col, 0)
        pltpu.sync_copy(acc, y_hbm.at[pl.ds(t0, TPC)])
        return carry

    lax.fori_loop(0, TPW // TPC, chunk, 0)


def _run_combine(ys, pos_flat, b2):
    mesh = plsc.VectorSubcoreMesh(core_axis_name="c", subcore_axis_name="s")
    f = functools.partial(
        pl.kernel,
        out_type=jax.ShapeDtypeStruct((T, IN_F), jnp.float32),
        mesh=mesh,
        scratch_types=[
            pltpu.VMEM((TPC * TOPK, IN_F), jnp.float32),
            pltpu.VMEM((TPC, IN_F), jnp.float32),
            pltpu.VMEM((TPC * TOPK,), jnp.int32),
            pltpu.VMEM((IN_F,), jnp.float32),
            pltpu.SemaphoreType.DMA,
        ],
    )(_combine_body)
    return f(ys, pos_flat, b2)


def kernel(x, Wr, br, W1, b1, W2, b2):
    logits = x @ Wr.T + br[None, :]
    prob = jax.nn.softmax(logits, axis=-1)

    pos_tok, pos_t, te, xi, tv = _run_meta(prob)
    te = te.reshape(NTP)
    xi = xi.reshape(NTP)
    tv = tv.reshape(NTP)

    xs = _run_scatter(x, pos_t)

    b1r = b1.reshape(NB, 1, BLK)
    ys = _run_ffn(te, xi, tv, xs, W1, b1r, W2)

    pos_flat = pos_tok.reshape(T * TOPK)
    y = _run_combine(ys, pos_flat, b2)
    return y
